# baseline (speedup 1.0000x reference)
import functools

import jax
import jax.numpy as jnp
from jax import lax
from jax.experimental import pallas as pl
from jax.experimental.pallas import tpu as pltpu
from jax.experimental.pallas import tpu_sc as plsc

N_NOTES = 1000
N_RATERS = 1000
NDIM = 64
BATCH = 16384

NC = 2
NS = 16
NW = NC * NS
BPW = BATCH // NW
JCH = 8
CH = BPW // JCH
GRP = 16
NGRP = BPW // GRP
SCALE = 1.0 / (NDIM ** 0.5)

_mesh = plsc.VectorSubcoreMesh(core_axis_name="c", subcore_axis_name="s")


@functools.partial(
    pl.kernel,
    out_type=jax.ShapeDtypeStruct((BATCH,), jnp.float32),
    mesh=_mesh,
    compiler_params=pltpu.CompilerParams(
        needs_layout_passes=False, use_tc_tiling_on_sc=False),
    scratch_types=[
        pltpu.VMEM((JCH, CH), jnp.int32),
        pltpu.VMEM((JCH, CH), jnp.int32),
        pltpu.VMEM((BPW, NDIM), jnp.float32),
        pltpu.VMEM((BPW, NDIM), jnp.float32),
        pltpu.VMEM((N_NOTES,), jnp.float32),
        pltpu.VMEM((N_RATERS,), jnp.float32),
        pltpu.VMEM((N_RATERS,), jnp.float32),
        pltpu.VMEM((BPW,), jnp.float32),
        pltpu.SemaphoreType.DMA,
        pltpu.SemaphoreType.DMA,
    ] + [pltpu.SemaphoreType.DMA] * JCH,
)
def _mf_kernel(notes_h, raters_h, nemb_h, remb_h, nb_h, rb_h, rr_h,
               out_h, idx_n, idx_r, nrows, rrows, nb_v, rb_v, rr_v,
               out_v, sem_i, sem_b, *csems):
    wid = lax.axis_index("s") * NC + lax.axis_index("c")
    base = wid * BPW
    h_in = pltpu.async_copy(notes_h.at[wid], idx_n, sem_i)
    h_ir = pltpu.async_copy(raters_h.at[wid], idx_r, sem_i)
    h_b = [pltpu.async_copy(nb_h, nb_v, sem_b),
           pltpu.async_copy(rb_h, rb_v, sem_b),
           pltpu.async_copy(rr_h, rr_v, sem_b)]
    h_in.wait()
    h_ir.wait()

    for j in range(JCH):
        pltpu.async_copy(nemb_h.at[idx_n.at[j]],
                         nrows.at[pl.ds(j * CH, CH)], csems[j])
        pltpu.async_copy(remb_h.at[idx_r.at[j]],
                         rrows.at[pl.ds(j * CH, CH)], csems[j])
    for h in h_b:
        h.wait()

    lane = lax.iota(jnp.int32, 16)

    GPC = NGRP // JCH

    def do_group(g):
        jj = g // GPC
        qq = g % GPC
        row0 = g * GRP
        nvec = idx_n[jj, pl.ds(qq * GRP, GRP)]
        rvec = idx_r[jj, pl.ds(qq * GRP, GRP)]
        vs = []
        for r in range(GRP):
            row = row0 + r
            ps = [nrows[row, pl.ds(k * 16, 16)] * rrows[row, pl.ds(k * 16, 16)]
                  for k in range(NDIM // 16)]
            vs.append((ps[0] + ps[1]) + (ps[2] + ps[3]))
        for s in (1, 2, 4, 8):
            nxt = []
            for i in range(len(vs) // 2):
                a, b = vs[2 * i], vs[2 * i + 1]
                pidx = lane ^ s
                nxt.append(jnp.where((lane & s) == 0, a + a[pidx], b + b[pidx]))
            vs = nxt
        acc = vs[0]
        nb = plsc.load_gather(nb_v, [nvec])
        rb = plsc.load_gather(rb_v, [rvec])
        rr = plsc.load_gather(rr_v, [rvec])
        pred = acc * SCALE + nb * rr + rb
        out_v[pl.ds(row0, GRP)] = pred

    def pair(p, _):
        for j in range(JCH):
            @pl.when(p == j * GPC // 2)
            def _(j=j):
                pltpu.make_async_copy(
                    nemb_h.at[idx_n.at[j]],
                    nrows.at[pl.ds(j * CH, CH)], csems[j]).wait()
                pltpu.make_async_copy(
                    remb_h.at[idx_r.at[j]],
                    rrows.at[pl.ds(j * CH, CH)], csems[j]).wait()
        do_group(p * 2)
        do_group(p * 2 + 1)
        return 0

    lax.fori_loop(0, NGRP // 2, pair, 0)

    pltpu.sync_copy(out_v, out_h.at[pl.ds(base, BPW)])


def kernel(notes, raters, noteEmb, raterEmb, noteBias, raterBias, raterRep,
           globalBias):
    notes_r = notes.astype(jnp.int32).reshape(NW, JCH, CH)
    raters_r = raters.astype(jnp.int32).reshape(NW, JCH, CH)
    nb = noteBias.reshape(N_NOTES)
    rb = raterBias.reshape(N_RATERS) + globalBias.astype(jnp.float32)
    rr = raterRep.reshape(N_RATERS)
    out = _mf_kernel(notes_r, raters_r, noteEmb, raterEmb, nb, rb, rr)
    return out.reshape(BATCH, 1)


# speedup vs baseline: 1.0500x; 1.0500x over previous
import functools

import jax
import jax.numpy as jnp
from jax import lax
from jax.experimental import pallas as pl
from jax.experimental.pallas import tpu as pltpu
from jax.experimental.pallas import tpu_sc as plsc

N_NOTES = 1000
N_RATERS = 1000
NDIM = 64
BATCH = 16384

NC = 2
NS = 16
NW = NC * NS
BPW = BATCH // NW
JCH = 8
CH = BPW // JCH
GRP = 16
NGRP = BPW // GRP
SCALE = 1.0 / (NDIM ** 0.5)

_mesh = plsc.VectorSubcoreMesh(core_axis_name="c", subcore_axis_name="s")


@functools.partial(
    pl.kernel,
    out_type=jax.ShapeDtypeStruct((BATCH,), jnp.float32),
    mesh=_mesh,
    compiler_params=pltpu.CompilerParams(
        needs_layout_passes=False, use_tc_tiling_on_sc=False,
        disable_bounds_checks=True, disable_semaphore_checks=True,
        skip_device_barrier=True),
    scratch_types=[
        pltpu.VMEM((JCH, CH), jnp.int32),
        pltpu.VMEM((JCH, CH), jnp.int32),
        pltpu.VMEM((BPW, NDIM), jnp.float32),
        pltpu.VMEM((BPW, NDIM), jnp.float32),
        pltpu.VMEM((N_NOTES,), jnp.float32),
        pltpu.VMEM((N_RATERS,), jnp.float32),
        pltpu.VMEM((N_RATERS,), jnp.float32),
        pltpu.VMEM((BPW,), jnp.float32),
        pltpu.SemaphoreType.DMA,
        pltpu.SemaphoreType.DMA,
    ] + [pltpu.SemaphoreType.DMA] * JCH,
)
def _mf_kernel(notes_h, raters_h, nemb_h, remb_h, nb_h, rb_h, rr_h,
               out_h, idx_n, idx_r, nrows, rrows, nb_v, rb_v, rr_v,
               out_v, sem_i, sem_b, *csems):
    wid = lax.axis_index("s") * NC + lax.axis_index("c")
    base = wid * BPW
    h_in = pltpu.async_copy(notes_h.at[wid], idx_n, sem_i)
    h_ir = pltpu.async_copy(raters_h.at[wid], idx_r, sem_i)
    h_b = [pltpu.async_copy(nb_h, nb_v, sem_b),
           pltpu.async_copy(rb_h, rb_v, sem_b),
           pltpu.async_copy(rr_h, rr_v, sem_b)]
    h_in.wait()
    h_ir.wait()

    for j in range(JCH):
        pltpu.async_copy(nemb_h.at[idx_n.at[j]],
                         nrows.at[pl.ds(j * CH, CH)], csems[j])
        pltpu.async_copy(remb_h.at[idx_r.at[j]],
                         rrows.at[pl.ds(j * CH, CH)], csems[j])
    for h in h_b:
        h.wait()

    lane = lax.iota(jnp.int32, 16)

    def group(g, _):
        for j in range(JCH):
            @pl.when(g == j * (NGRP // JCH))
            def _(j=j):
                pltpu.make_async_copy(
                    nemb_h.at[idx_n.at[j]],
                    nrows.at[pl.ds(j * CH, CH)], csems[j]).wait()
                pltpu.make_async_copy(
                    remb_h.at[idx_r.at[j]],
                    rrows.at[pl.ds(j * CH, CH)], csems[j]).wait()

        jj = g // (NGRP // JCH)
        qq = g % (NGRP // JCH)
        row0 = g * GRP
        nvec = idx_n[jj, pl.ds(qq * GRP, GRP)]
        rvec = idx_r[jj, pl.ds(qq * GRP, GRP)]
        vs = []
        for r in range(GRP):
            row = row0 + r
            ps = [nrows[row, pl.ds(k * 16, 16)] * rrows[row, pl.ds(k * 16, 16)]
                  for k in range(NDIM // 16)]
            vs.append((ps[0] + ps[1]) + (ps[2] + ps[3]))
        for s in (1, 2, 4, 8):
            nxt = []
            for i in range(len(vs) // 2):
                a, b = vs[2 * i], vs[2 * i + 1]
                pidx = lane ^ s
                nxt.append(jnp.where((lane & s) == 0, a + a[pidx], b + b[pidx]))
            vs = nxt
        acc = vs[0]
        nb = plsc.load_gather(nb_v, [nvec])
        rb = plsc.load_gather(rb_v, [rvec])
        rr = plsc.load_gather(rr_v, [rvec])
        pred = acc * SCALE + nb * rr + rb
        out_v[pl.ds(row0, GRP)] = pred
        return 0

    lax.fori_loop(0, NGRP, group, 0)

    pltpu.sync_copy(out_v, out_h.at[pl.ds(base, BPW)])


def kernel(notes, raters, noteEmb, raterEmb, noteBias, raterBias, raterRep,
           globalBias):
    notes_r = notes.astype(jnp.int32).reshape(NW, JCH, CH)
    raters_r = raters.astype(jnp.int32).reshape(NW, JCH, CH)
    nb = noteBias.reshape(N_NOTES)
    rb = raterBias.reshape(N_RATERS) + globalBias.astype(jnp.float32)
    rr = raterRep.reshape(N_RATERS)
    out = _mf_kernel(notes_r, raters_r, noteEmb, raterEmb, nb, rb, rr)
    return out.reshape(BATCH, 1)


# speedup vs baseline: 1.1082x; 1.0554x over previous
import functools

import jax
import jax.numpy as jnp
from jax import lax
from jax.experimental import pallas as pl
from jax.experimental.pallas import tpu as pltpu
from jax.experimental.pallas import tpu_sc as plsc

N_NOTES = 1000
N_RATERS = 1000
NDIM = 64
BATCH = 16384

NC = 2
NS = 16
NW = NC * NS
BPW = BATCH // NW
JCH = 8
CH = BPW // JCH
GRP = 16
NGRP = BPW // GRP
SCALE = 1.0 / (NDIM ** 0.5)

_mesh = plsc.VectorSubcoreMesh(core_axis_name="c", subcore_axis_name="s")


@functools.partial(
    pl.kernel,
    out_type=jax.ShapeDtypeStruct((BATCH,), jnp.float32),
    mesh=_mesh,
    compiler_params=pltpu.CompilerParams(
        needs_layout_passes=False, use_tc_tiling_on_sc=False,
        disable_bounds_checks=True, disable_semaphore_checks=True,
        skip_device_barrier=True),
    scratch_types=[
        pltpu.VMEM((JCH, CH), jnp.int32),
        pltpu.VMEM((JCH, CH), jnp.int32),
        pltpu.VMEM((BPW, NDIM // 2), jnp.int32),
        pltpu.VMEM((BPW, NDIM // 2), jnp.int32),
        pltpu.VMEM((N_NOTES,), jnp.float32),
        pltpu.VMEM((N_RATERS,), jnp.float32),
        pltpu.VMEM((N_RATERS,), jnp.float32),
        pltpu.VMEM((BPW,), jnp.float32),
        pltpu.SemaphoreType.DMA,
        pltpu.SemaphoreType.DMA,
    ] + [pltpu.SemaphoreType.DMA] * JCH,
)
def _mf_kernel(notes_h, raters_h, nemb_h, remb_h, nb_h, rb_h, rr_h,
               out_h, idx_n, idx_r, nrows, rrows, nb_v, rb_v, rr_v,
               out_v, sem_i, sem_b, *csems):
    wid = lax.axis_index("s") * NC + lax.axis_index("c")
    base = wid * BPW
    h_in = pltpu.async_copy(notes_h.at[wid], idx_n, sem_i)
    h_ir = pltpu.async_copy(raters_h.at[wid], idx_r, sem_i)
    h_b = [pltpu.async_copy(nb_h, nb_v, sem_b),
           pltpu.async_copy(rb_h, rb_v, sem_b),
           pltpu.async_copy(rr_h, rr_v, sem_b)]
    h_in.wait()
    h_ir.wait()

    for j in range(JCH):
        pltpu.async_copy(nemb_h.at[idx_n.at[j]],
                         nrows.at[pl.ds(j * CH, CH)], csems[j])
        pltpu.async_copy(remb_h.at[idx_r.at[j]],
                         rrows.at[pl.ds(j * CH, CH)], csems[j])
    for h in h_b:
        h.wait()

    lane = lax.iota(jnp.int32, 16)

    def group(g, _):
        for j in range(JCH):
            @pl.when(g == j * (NGRP // JCH))
            def _(j=j):
                pltpu.make_async_copy(
                    nemb_h.at[idx_n.at[j]],
                    nrows.at[pl.ds(j * CH, CH)], csems[j]).wait()
                pltpu.make_async_copy(
                    remb_h.at[idx_r.at[j]],
                    rrows.at[pl.ds(j * CH, CH)], csems[j]).wait()

        jj = g // (NGRP // JCH)
        qq = g % (NGRP // JCH)
        row0 = g * GRP
        nvec = idx_n[jj, pl.ds(qq * GRP, GRP)]
        rvec = idx_r[jj, pl.ds(qq * GRP, GRP)]
        vs = []
        for r in range(GRP):
            row = row0 + r
            ps = []
            for k in range(NDIM // 32):
                nw = plsc.bitcast(nrows[row, pl.ds(k * 16, 16)], jnp.bfloat16)
                rw = plsc.bitcast(rrows[row, pl.ds(k * 16, 16)], jnp.bfloat16)
                nlo, nhi = plsc.unpack(nw, format=plsc.PackFormat.INTERLEAVED)
                rlo, rhi = plsc.unpack(rw, format=plsc.PackFormat.INTERLEAVED)
                ps.append(nlo * rlo + nhi * rhi)
            vs.append(ps[0] + ps[1])
        for s in (1, 2, 4, 8):
            nxt = []
            for i in range(len(vs) // 2):
                a, b = vs[2 * i], vs[2 * i + 1]
                pidx = lane ^ s
                nxt.append(jnp.where((lane & s) == 0, a + a[pidx], b + b[pidx]))
            vs = nxt
        acc = vs[0]
        nb = plsc.load_gather(nb_v, [nvec])
        rb = plsc.load_gather(rb_v, [rvec])
        rr = plsc.load_gather(rr_v, [rvec])
        pred = acc * SCALE + nb * rr + rb
        out_v[pl.ds(row0, GRP)] = pred
        return 0

    lax.fori_loop(0, NGRP, group, 0)

    pltpu.sync_copy(out_v, out_h.at[pl.ds(base, BPW)])


def _pack_bf16(tab):
    b = tab.astype(jnp.bfloat16).reshape(tab.shape[0], NDIM // 2, 2)
    return lax.bitcast_convert_type(b, jnp.int32)


def kernel(notes, raters, noteEmb, raterEmb, noteBias, raterBias, raterRep,
           globalBias):
    notes_r = notes.astype(jnp.int32).reshape(NW, JCH, CH)
    raters_r = raters.astype(jnp.int32).reshape(NW, JCH, CH)
    nb = noteBias.reshape(N_NOTES)
    rb = raterBias.reshape(N_RATERS) + globalBias.astype(jnp.float32)
    rr = raterRep.reshape(N_RATERS)
    out = _mf_kernel(notes_r, raters_r, _pack_bf16(noteEmb),
                     _pack_bf16(raterEmb), nb, rb, rr)
    return out.reshape(BATCH, 1)


# speedup vs baseline: 1.1124x; 1.0038x over previous
import functools

import jax
import jax.numpy as jnp
from jax import lax
from jax.experimental import pallas as pl
from jax.experimental.pallas import tpu as pltpu
from jax.experimental.pallas import tpu_sc as plsc

N_NOTES = 1000
N_RATERS = 1000
NDIM = 64
BATCH = 16384

NC = 2
NS = 16
NW = NC * NS
BPW = BATCH // NW
JCH = 8
CH = BPW // JCH
GRP = 16
NGRP = BPW // GRP
SCALE = 1.0 / (NDIM ** 0.5)

_mesh = plsc.VectorSubcoreMesh(core_axis_name="c", subcore_axis_name="s")


@functools.partial(
    pl.kernel,
    out_type=jax.ShapeDtypeStruct((BATCH,), jnp.float32),
    mesh=_mesh,
    compiler_params=pltpu.CompilerParams(
        needs_layout_passes=False, use_tc_tiling_on_sc=False,
        disable_bounds_checks=True, disable_semaphore_checks=True,
        skip_device_barrier=True),
    scratch_types=[
        pltpu.VMEM((JCH, CH), jnp.int32),
        pltpu.VMEM((JCH, CH), jnp.int32),
        pltpu.VMEM((BPW, NDIM // 2), jnp.int32),
        pltpu.VMEM((BPW, NDIM // 2), jnp.int32),
        pltpu.VMEM((N_NOTES,), jnp.float32),
        pltpu.VMEM((N_RATERS,), jnp.float32),
        pltpu.VMEM((N_RATERS,), jnp.float32),
        pltpu.VMEM((BPW,), jnp.float32),
        pltpu.SemaphoreType.DMA,
        pltpu.SemaphoreType.DMA,
    ] + [pltpu.SemaphoreType.DMA] * JCH,
)
def _mf_kernel(notes_h, raters_h, nemb_h, remb_h, nb_h, rb_h, rr_h,
               out_h, idx_n, idx_r, nrows, rrows, nb_v, rb_v, rr_v,
               out_v, sem_i, sem_b, *csems):
    wid = lax.axis_index("s") * NC + lax.axis_index("c")
    base = wid * BPW
    h_i = []
    for j in range(JCH):
        h_i.append(pltpu.async_copy(notes_h.at[wid, j], idx_n.at[j], sem_i))
        h_i.append(pltpu.async_copy(raters_h.at[wid, j], idx_r.at[j], sem_i))
    h_b = [pltpu.async_copy(nb_h, nb_v, sem_b),
           pltpu.async_copy(rb_h, rb_v, sem_b),
           pltpu.async_copy(rr_h, rr_v, sem_b)]

    for j in range(JCH):
        h_i[2 * j].wait()
        h_i[2 * j + 1].wait()
        pltpu.async_copy(nemb_h.at[idx_n.at[j]],
                         nrows.at[pl.ds(j * CH, CH)], csems[j])
        pltpu.async_copy(remb_h.at[idx_r.at[j]],
                         rrows.at[pl.ds(j * CH, CH)], csems[j])
    for h in h_b:
        h.wait()

    lane = lax.iota(jnp.int32, 16)

    def group(g, _):
        for j in range(JCH):
            @pl.when(g == j * (NGRP // JCH))
            def _(j=j):
                pltpu.make_async_copy(
                    nemb_h.at[idx_n.at[j]],
                    nrows.at[pl.ds(j * CH, CH)], csems[j]).wait()
                pltpu.make_async_copy(
                    remb_h.at[idx_r.at[j]],
                    rrows.at[pl.ds(j * CH, CH)], csems[j]).wait()

        jj = g // (NGRP // JCH)
        qq = g % (NGRP // JCH)
        row0 = g * GRP
        nvec = idx_n[jj, pl.ds(qq * GRP, GRP)]
        rvec = idx_r[jj, pl.ds(qq * GRP, GRP)]
        vs = []
        for r in range(GRP):
            row = row0 + r
            ps = []
            for k in range(NDIM // 32):
                nw = plsc.bitcast(nrows[row, pl.ds(k * 16, 16)], jnp.bfloat16)
                rw = plsc.bitcast(rrows[row, pl.ds(k * 16, 16)], jnp.bfloat16)
                nlo, nhi = plsc.unpack(nw, format=plsc.PackFormat.INTERLEAVED)
                rlo, rhi = plsc.unpack(rw, format=plsc.PackFormat.INTERLEAVED)
                ps.append(nlo * rlo + nhi * rhi)
            vs.append(ps[0] + ps[1])
        for s in (1, 2, 4, 8):
            nxt = []
            for i in range(len(vs) // 2):
                a, b = vs[2 * i], vs[2 * i + 1]
                pidx = lane ^ s
                nxt.append(jnp.where((lane & s) == 0, a + a[pidx], b + b[pidx]))
            vs = nxt
        acc = vs[0]
        nb = plsc.load_gather(nb_v, [nvec])
        rb = plsc.load_gather(rb_v, [rvec])
        rr = plsc.load_gather(rr_v, [rvec])
        pred = acc + nb * rr + rb
        out_v[pl.ds(row0, GRP)] = pred
        return 0

    lax.fori_loop(0, NGRP, group, 0)

    pltpu.sync_copy(out_v, out_h.at[pl.ds(base, BPW)])


def _pack_bf16(tab, scale=1.0):
    b = (tab * scale).astype(jnp.bfloat16).reshape(tab.shape[0], NDIM // 2, 2)
    return lax.bitcast_convert_type(b, jnp.int32)


def kernel(notes, raters, noteEmb, raterEmb, noteBias, raterBias, raterRep,
           globalBias):
    notes_r = notes.astype(jnp.int32).reshape(NW, JCH, CH)
    raters_r = raters.astype(jnp.int32).reshape(NW, JCH, CH)
    nb = noteBias.reshape(N_NOTES)
    rb = raterBias.reshape(N_RATERS) + globalBias.astype(jnp.float32)
    rr = raterRep.reshape(N_RATERS)
    out = _mf_kernel(notes_r, raters_r, _pack_bf16(noteEmb, SCALE),
                     _pack_bf16(raterEmb), nb, rb, rr)
    return out.reshape(BATCH, 1)
